# baseline (speedup 1.0000x reference)
import functools

import jax
import jax.numpy as jnp
from jax import lax
from jax.experimental import pallas as pl
from jax.experimental.pallas import tpu as pltpu
from jax.experimental.pallas import tpu_sc as plsc

_B, _T, _D = 4, 4096, 1024
_A0, _A1 = 64, 64
_NC, _NS = 2, 16
_NW = _NC * _NS
_IPW = _A0 // _NW
_HJ = _A1 // 2


def _sc_body(w0_hbm, w1_hbm, out_hbm, w0_v, w1_v, obuf, osem):
    wid = lax.axis_index("s") * _NC + lax.axis_index("c")
    pltpu.sync_copy(w0_hbm.at[pl.ds(wid * _IPW, _IPW), :], w0_v)

    def compute_half(ii, h):
        def dloop(dc, _):
            va = w0_v[ii, pl.ds(dc * 16, 16)]
            for j in range(_HJ):
                obuf[h * _HJ + j, pl.ds(dc * 16, 16)] = (
                    va + w1_v[j, pl.ds(dc * 16, 16)])
            return 0
        lax.fori_loop(0, _D // 16, dloop, 0)

    for ii in range(_IPW):
        if ii > 0:
            pt0 = (wid * _IPW + ii - 1) * _A1
            for b in range(_B):
                pltpu.make_async_copy(
                    obuf,
                    out_hbm.at[pl.ds(b * _T + pt0, _A1), :],
                    osem.at[b],
                ).wait()
        for h in range(2):
            pltpu.sync_copy(w1_hbm.at[pl.ds(h * _HJ, _HJ), :], w1_v)
            compute_half(ii, h)
        t0 = (wid * _IPW + ii) * _A1
        for b in range(_B):
            pltpu.make_async_copy(
                obuf,
                out_hbm.at[pl.ds(b * _T + t0, _A1), :],
                osem.at[b],
            ).start()

    pt0 = (wid * _IPW + _IPW - 1) * _A1
    for b in range(_B):
        pltpu.make_async_copy(
            obuf,
            out_hbm.at[pl.ds(b * _T + pt0, _A1), :],
            osem.at[b],
        ).wait()


_sc_call = functools.partial(
    pl.kernel,
    mesh=plsc.VectorSubcoreMesh(core_axis_name="c", subcore_axis_name="s"),
    out_type=jax.ShapeDtypeStruct((_B * _T, _D), jnp.float32),
    scratch_types=[
        pltpu.VMEM((_IPW, _D), jnp.float32),
        pltpu.VMEM((_HJ, _D), jnp.float32),
        pltpu.VMEM((_A1, _D), jnp.float32),
        pltpu.SemaphoreType.DMA((_B,)),
    ],
)(_sc_body)


def kernel(x, w0, w1):
    del x
    out = _sc_call(w0.reshape(_A0, _D), w1.reshape(_A1, _D))
    return out.reshape(_B, _T, _D)


# speedup vs baseline: 3.2291x; 3.2291x over previous
import jax
import jax.numpy as jnp
from jax.experimental import pallas as pl
from jax.experimental.pallas import tpu as pltpu

_B, _T, _D = 4, 4096, 1024
_A0, _A1 = 64, 64

_RPB = 4
_TB = _RPB * _A1
_NBLK = _A0 // _RPB
_NBUF = 3


def _body(w0_ref, w1_ref, out_ref, scr_ref, sem_ref):
    k = pl.program_id(0)
    slot = jax.lax.rem(k, _NBUF)

    @pl.when(k >= _NBUF)
    def _():
        for b in range(_B):
            pltpu.make_async_copy(
                scr_ref.at[slot],
                out_ref.at[b, pl.ds((k - _NBUF) * _TB, _TB), :],
                sem_ref.at[slot, b],
            ).wait()

    rows = w0_ref[0, :, 0, :]
    tile = w1_ref[0, 0, :, :]
    s = rows[:, None, :] + tile[None, :, :]
    scr_ref[slot] = s.reshape(_TB, _D)

    for b in range(_B):
        pltpu.make_async_copy(
            scr_ref.at[slot],
            out_ref.at[b, pl.ds(k * _TB, _TB), :],
            sem_ref.at[slot, b],
        ).start()

    @pl.when(k == _NBLK - 1)
    def _():
        for back in range(_NBUF - 1, -1, -1):
            kk = k - back
            sl = jax.lax.rem(kk, _NBUF)
            for b in range(_B):
                pltpu.make_async_copy(
                    scr_ref.at[sl],
                    out_ref.at[b, pl.ds(kk * _TB, _TB), :],
                    sem_ref.at[sl, b],
                ).wait()


def kernel(x, w0, w1):
    del x
    out = pl.pallas_call(
        _body,
        grid=(_NBLK,),
        in_specs=[
            pl.BlockSpec((1, _RPB, 1, _D), lambda k: (0, k, 0, 0)),
            pl.BlockSpec((1, 1, _A1, _D), lambda k: (0, 0, 0, 0)),
        ],
        out_specs=pl.BlockSpec(memory_space=pltpu.MemorySpace.HBM),
        out_shape=jax.ShapeDtypeStruct((_B, _T, _D), jnp.float32),
        scratch_shapes=[
            pltpu.VMEM((_NBUF, _TB, _D), jnp.float32),
            pltpu.SemaphoreType.DMA((_NBUF, _B)),
        ],
    )(w0, w1)
    return out


# speedup vs baseline: 3.2868x; 1.0179x over previous
import jax
import jax.numpy as jnp
from jax.experimental import pallas as pl
from jax.experimental.pallas import tpu as pltpu

_B, _T, _D = 4, 4096, 1024
_A0, _A1 = 64, 64

_RPB = 8
_TB = _RPB * _A1
_NBLK = _A0 // _RPB
_NBUF = 2


def _body(w0_ref, w1_ref, out_ref, scr_ref, sem_ref):
    k = pl.program_id(0)
    slot = jax.lax.rem(k, _NBUF)

    @pl.when(k >= _NBUF)
    def _():
        for b in range(_B):
            pltpu.make_async_copy(
                scr_ref.at[slot],
                out_ref.at[b, pl.ds((k - _NBUF) * _TB, _TB), :],
                sem_ref.at[slot, b],
            ).wait()

    rows = w0_ref[0, :, 0, :]
    tile = w1_ref[0, 0, :, :]
    s = rows[:, None, :] + tile[None, :, :]
    scr_ref[slot] = s.reshape(_TB, _D)

    for b in range(_B):
        pltpu.make_async_copy(
            scr_ref.at[slot],
            out_ref.at[b, pl.ds(k * _TB, _TB), :],
            sem_ref.at[slot, b],
        ).start()

    @pl.when(k == _NBLK - 1)
    def _():
        for back in range(_NBUF - 1, -1, -1):
            kk = k - back
            sl = jax.lax.rem(kk, _NBUF)
            for b in range(_B):
                pltpu.make_async_copy(
                    scr_ref.at[sl],
                    out_ref.at[b, pl.ds(kk * _TB, _TB), :],
                    sem_ref.at[sl, b],
                ).wait()


def kernel(x, w0, w1):
    del x
    out = pl.pallas_call(
        _body,
        grid=(_NBLK,),
        in_specs=[
            pl.BlockSpec((1, _RPB, 1, _D), lambda k: (0, k, 0, 0)),
            pl.BlockSpec((1, 1, _A1, _D), lambda k: (0, 0, 0, 0)),
        ],
        out_specs=pl.BlockSpec(memory_space=pltpu.MemorySpace.HBM),
        out_shape=jax.ShapeDtypeStruct((_B, _T, _D), jnp.float32),
        scratch_shapes=[
            pltpu.VMEM((_NBUF, _TB, _D), jnp.float32),
            pltpu.SemaphoreType.DMA((_NBUF, _B)),
        ],
    )(w0, w1)
    return out
